# Initial kernel scaffold; baseline (speedup 1.0000x reference)
import jax
import jax.numpy as jnp
from jax.experimental import pallas as pl


def kernel(fe, gemm_edges, W, b):
    raise NotImplementedError("write your pallas kernel here")


# speedup vs baseline: 2.4588x; 2.4588x over previous
import functools

import jax
import jax.numpy as jnp
from jax import lax
from jax.experimental import pallas as pl
from jax.experimental.pallas import tpu as pltpu
from jax.experimental.pallas import tpu_sc as plsc

EPS = 1e-5


def _transpose_fe(fe2):
    C, E = fe2.shape
    Et = 1280
    grid = (E // Et,)

    def body(x_ref, o_ref):
        o_ref[...] = x_ref[...].T

    return pl.pallas_call(
        body,
        grid=grid,
        in_specs=[pl.BlockSpec((C, Et), lambda i: (0, i))],
        out_specs=pl.BlockSpec((Et, C), lambda i: (i, 0)),
        out_shape=jax.ShapeDtypeStruct((E, C), jnp.float32),
    )(fe2)


def _sc_gather(feT, idx_flat):
    E, C = feT.shape
    NI = idx_flat.shape[1]
    GW = 128

    mesh = plsc.VectorSubcoreMesh(
        core_axis_name="core", subcore_axis_name="subcore", num_cores=2
    )

    @functools.partial(
        pl.kernel,
        out_type=jax.ShapeDtypeStruct((NI, C), jnp.float32),
        mesh=mesh,
    )
    def gather_kernel(x_hbm, i_hbm, o_hbm):
        def body(i_vmem, o_vmem):
            pltpu.sync_copy(x_hbm.at[i_vmem.at[0]], o_vmem)

        pltpu.emit_pipeline(
            body,
            grid=(NI // GW,),
            in_specs=[pl.BlockSpec((1, GW), index_map=lambda i: (0, i))],
            out_specs=[pl.BlockSpec((GW, C), index_map=lambda i: (i, 0))],
            core_axis_name=("core", "subcore"),
            dimension_semantics=(pltpu.PARALLEL,),
        )(i_hbm, o_hbm)

    return gather_kernel(feT, idx_flat)


def _conv_relu_stats(G2, feT, Wf, bb):
    E, C4 = G2.shape
    C = C4 // 4
    CO = Wf.shape[1]
    Eb = 640
    grid = (E // Eb,)

    def body(g_ref, s_ref, w_ref, b_ref, y_ref, sum_ref):
        g = g_ref[...]
        f1 = g[:, 0:C]
        f2 = g[:, C : 2 * C]
        f3 = g[:, 2 * C : 3 * C]
        f4 = g[:, 3 * C : 4 * C]
        x1 = f1 + f3
        x2 = f2 + f4
        x3 = jnp.abs(f1 - f3)
        x4 = jnp.abs(f2 - f4)
        X = jnp.concatenate([s_ref[...], x1, x2, x3, x4], axis=1)
        y = jnp.dot(
            X.astype(jnp.bfloat16),
            w_ref[...].astype(jnp.bfloat16),
            preferred_element_type=jnp.float32,
        )
        y = y + b_ref[0:1, :]
        y = jnp.maximum(y, 0.0)
        y_ref[...] = y
        s1 = jnp.sum(y, axis=0, keepdims=True)
        s2 = jnp.sum(y * y, axis=0, keepdims=True)
        blk = jnp.concatenate(
            [s1, s2, jnp.zeros((6, CO), jnp.float32)], axis=0
        )
        i = pl.program_id(0)

        @pl.when(i == 0)
        def _():
            sum_ref[...] = blk

        @pl.when(i > 0)
        def _():
            sum_ref[...] += blk

    return pl.pallas_call(
        body,
        grid=grid,
        in_specs=[
            pl.BlockSpec((Eb, C4), lambda i: (i, 0)),
            pl.BlockSpec((Eb, C), lambda i: (i, 0)),
            pl.BlockSpec((5 * C, CO), lambda i: (0, 0)),
            pl.BlockSpec((8, CO), lambda i: (0, 0)),
        ],
        out_specs=[
            pl.BlockSpec((Eb, CO), lambda i: (i, 0)),
            pl.BlockSpec((8, CO), lambda i: (0, 0)),
        ],
        out_shape=[
            jax.ShapeDtypeStruct((E, CO), jnp.float32),
            jax.ShapeDtypeStruct((8, CO), jnp.float32),
        ],
        compiler_params=pltpu.CompilerParams(
            dimension_semantics=("arbitrary",)
        ),
    )(G2, feT, Wf, bb)


def _norm_transpose(y, sums):
    E, CO = y.shape
    Eb = 1280
    grid = (E // Eb,)
    inv_e = 1.0 / E

    def body(y_ref, s_ref, o_ref):
        mu = s_ref[0:1, :] * inv_e
        var = s_ref[1:2, :] * inv_e - mu * mu
        inv = lax.rsqrt(var + EPS)
        yn = (y_ref[...] - mu) * inv
        o_ref[...] = yn.T

    return pl.pallas_call(
        body,
        grid=grid,
        in_specs=[
            pl.BlockSpec((Eb, CO), lambda i: (i, 0)),
            pl.BlockSpec((8, CO), lambda i: (0, 0)),
        ],
        out_specs=pl.BlockSpec((CO, Eb), lambda i: (0, i)),
        out_shape=jax.ShapeDtypeStruct((CO, E), jnp.float32),
    )(y, sums)


def kernel(fe, gemm_edges, W, b):
    B, C, E = fe.shape
    CO = W.shape[0]
    fe2 = fe[0]
    idx_flat = gemm_edges[0].reshape(1, 4 * E)

    feT = _transpose_fe(fe2)
    G = _sc_gather(feT, idx_flat)
    G2 = G.reshape(E, 4 * C)

    Wf = jnp.transpose(W, (2, 1, 0)).reshape(5 * C, CO)
    bb = jnp.broadcast_to(b[None, :], (8, CO))

    y, sums = _conv_relu_stats(G2, feT, Wf, bb)
    out = _norm_transpose(y, sums)
    return out[None]
